# baseline (speedup 1.0000x reference)
import functools
import math

import jax
import jax.numpy as jnp
import numpy as np
from jax.experimental import pallas as pl

B, C, H, W = 4, 256, 128, 128
HW = H * W
K = int(0.9 * HW)
TT = 1024
NT = HW // TT
LOG1E4 = math.log(10000.0)


def _dense_body(cat_ref, cw_ref, cb_ref, mfwT_ref, mfb_ref, mlg_ref, mlb_ref,
                clswT_ref, clsb_ref, w1T_ref, b1_ref, w2T_ref, b2_ref,
                w3T_ref, b3_ref, pfwT_ref, pfb_ref, plg_ref, plb_ref, prop_ref,
                esu_ref, pos_ref, small_ref, meta_ref):
    t = pl.program_id(1)
    cat = cat_ref[0]
    esu_cm = jnp.dot(cw_ref[...], cat, preferred_element_type=jnp.float32)
    esu_cm = esu_cm + cb_ref[...]
    esu_t = esu_cm.T
    esu_ref[0] = esu_t

    ii = jax.lax.broadcasted_iota(jnp.int32, (TT, 1), 0)
    tok = t * TT + ii
    x = tok & (W - 1)
    y = tok >> 7
    valid = (x >= 1) & (x <= W - 2) & (y >= 1) & (y <= H - 2)
    om = jnp.where(valid, esu_t, 0.0)

    om1 = jnp.dot(om, mfwT_ref[...], preferred_element_type=jnp.float32) + mfb_ref[...]
    m = jnp.mean(om1, axis=1, keepdims=True)
    v = jnp.mean((om1 - m) ** 2, axis=1, keepdims=True)
    ln = (om1 - m) / jnp.sqrt(v + 1e-5) * mlg_ref[...] + mlb_ref[...]

    clsp = jnp.dot(ln, clswT_ref[...], preferred_element_type=jnp.float32) + clsb_ref[...]
    h1 = jnp.maximum(jnp.dot(ln, w1T_ref[...], preferred_element_type=jnp.float32) + b1_ref[...], 0.0)
    h2 = jnp.maximum(jnp.dot(h1, w2T_ref[...], preferred_element_type=jnp.float32) + b2_ref[...], 0.0)
    dpad = jnp.dot(h2, w3T_ref[...], preferred_element_type=jnp.float32) + b3_ref[...]

    prop = prop_ref[...]
    ux = dpad[:, 0:1] + prop[:, 0:1]
    uy = dpad[:, 1:2] + prop[:, 1:2]
    refx = jax.nn.sigmoid(ux)
    refy = jax.nn.sigmoid(uy)

    gx = refx * W - 0.5
    gy = refy * H - 0.5
    x0 = jnp.floor(gx)
    y0 = jnp.floor(gy)
    fx = gx - x0
    fy = gy - y0

    def nb(xi, yi):
        ok = (xi >= 0) & (xi < W) & (yi >= 0) & (yi < H)
        idx = jnp.clip(yi, 0, H - 1) * W + jnp.clip(xi, 0, W - 1)
        return idx, ok.astype(jnp.float32)

    i00, v00 = nb(x0, y0)
    i01, v01 = nb(x0 + 1, y0)
    i10, v10 = nb(x0, y0 + 1)
    i11, v11 = nb(x0 + 1, y0 + 1)
    w00 = (1 - fx) * (1 - fy) * v00
    w01 = fx * (1 - fy) * v01
    w10 = (1 - fx) * fy * v10
    w11 = fx * fy * v11
    meta_ref[0] = jnp.concatenate([i00, i01, i10, i11, w00, w01, w10, w11], axis=1)
    small_ref[0] = jnp.concatenate(
        [clsp[:, 0:1], clsp[:, 1:2], refx, refy, ux, uy, ux, uy], axis=1)

    k = jax.lax.broadcasted_iota(jnp.int32, (1, C), 1)
    j = (k & 127) >> 1
    tdiv = jnp.exp((2.0 * j.astype(jnp.float32) / 128.0) * LOG1E4)
    pxy = jnp.where(k < 128, refx * (2.0 * math.pi), refy * (2.0 * math.pi))
    ang = pxy / tdiv
    even = (k & 1) == 0
    posf = jnp.where(even, jnp.sin(ang), jnp.cos(ang))
    pn = jnp.dot(posf, pfwT_ref[...], preferred_element_type=jnp.float32) + pfb_ref[...]
    m2 = jnp.mean(pn, axis=1, keepdims=True)
    v2 = jnp.mean((pn - m2) ** 2, axis=1, keepdims=True)
    pos_ref[0] = (pn - m2) / jnp.sqrt(v2 + 1e-5) * plg_ref[...] + plb_ref[...]


def _prop_table():
    gy, gx = np.meshgrid(np.arange(H, dtype=np.float64), np.arange(W, dtype=np.float64), indexing='ij')
    px = ((gx + 0.5) / W).astype(np.float32)
    py = ((gy + 0.5) / H).astype(np.float32)
    lx = np.log(px / (1.0 - px)).astype(np.float32)
    ly = np.log(py / (1.0 - py)).astype(np.float32)
    valid = (px > 0.01) & (px < 0.99) & (py > 0.01) & (py < 0.99)
    lx = np.where(valid, lx, 1e6).astype(np.float32)
    ly = np.where(valid, ly, 1e6).astype(np.float32)
    z = np.zeros_like(lx)
    return np.stack([lx, ly, z, z, z, z, z, z], axis=-1).reshape(HW, 8)


def _dense_call(cat, conv_w, conv_b, mem_fc_w, mem_fc_b, mem_ln_g, mem_ln_b,
                cls_w, cls_b, mlp_w1, mlp_b1, mlp_w2, mlp_b2, mlp_w3, mlp_b3,
                pos_fc_w, pos_fc_b, pos_ln_g, pos_ln_b):
    clswT = jnp.zeros((C, 128), jnp.float32).at[:, :2].set(cls_w.T)
    clsbp = jnp.zeros((1, 128), jnp.float32).at[:, :2].set(cls_b)
    w3T = jnp.zeros((C, 128), jnp.float32).at[:, :2].set(mlp_w3.T)
    b3p = jnp.zeros((1, 128), jnp.float32).at[:, :2].set(mlp_b3)
    prop = jnp.asarray(_prop_table())

    full = lambda *shape: pl.BlockSpec(shape, lambda b, t: (0,) * len(shape))
    return pl.pallas_call(
        _dense_body,
        grid=(B, NT),
        in_specs=[
            pl.BlockSpec((1, 2 * C, TT), lambda b, t: (b, 0, t)),
            full(C, 2 * C), full(C, 1),
            full(C, C), full(1, C), full(1, C), full(1, C),
            full(C, 128), full(1, 128),
            full(C, C), full(1, C), full(C, C), full(1, C),
            full(C, 128), full(1, 128),
            full(C, C), full(1, C), full(1, C), full(1, C),
            pl.BlockSpec((TT, 8), lambda b, t: (t, 0)),
        ],
        out_specs=[
            pl.BlockSpec((1, TT, C), lambda b, t: (b, t, 0)),
            pl.BlockSpec((1, TT, C), lambda b, t: (b, t, 0)),
            pl.BlockSpec((1, TT, 8), lambda b, t: (b, t, 0)),
            pl.BlockSpec((1, TT, 8), lambda b, t: (b, t, 0)),
        ],
        out_shape=[
            jax.ShapeDtypeStruct((B, HW, C), jnp.float32),
            jax.ShapeDtypeStruct((B, HW, C), jnp.float32),
            jax.ShapeDtypeStruct((B, HW, 8), jnp.float32),
            jax.ShapeDtypeStruct((B, HW, 8), jnp.float32),
        ],
    )(cat, conv_w, conv_b.reshape(C, 1),
      mem_fc_w.T, mem_fc_b.reshape(1, C), mem_ln_g.reshape(1, C), mem_ln_b.reshape(1, C),
      clswT, clsbp,
      mlp_w1.T, mlp_b1.reshape(1, C), mlp_w2.T, mlp_b2.reshape(1, C),
      w3T, b3p,
      pos_fc_w.T, pos_fc_b.reshape(1, C), pos_ln_g.reshape(1, C), pos_ln_b.reshape(1, C),
      prop)


def _oracle_scores(cat4d, conv_w, conv_b, mem_fc_w, mem_fc_b, mem_ln_g, mem_ln_b,
                   cls_w, cls_b, invalid, ):
    esu = jnp.einsum('bchw,oc->bohw', cat4d, conv_w) + conv_b[None, :, None, None]
    output_memory = jnp.transpose(esu.reshape(B, C, HW), (0, 2, 1))
    output_memory = jnp.where(invalid, 0.0, output_memory)
    om = output_memory @ mem_fc_w.T + mem_fc_b
    m = jnp.mean(om, axis=-1, keepdims=True)
    v = jnp.var(om, axis=-1, keepdims=True)
    om = (om - m) / jnp.sqrt(v + 1e-5) * mem_ln_g + mem_ln_b
    cls = om @ cls_w.T + cls_b
    return jax.nn.softmax(cls, axis=-1)[..., 1]


def kernel(encode_src, feat_4x, mask, conv_w, conv_b, mem_fc_w, mem_fc_b,
           mem_ln_g, mem_ln_b, cls_w, cls_b, mlp_w1, mlp_b1, mlp_w2, mlp_b2,
           mlp_w3, mlp_b3, pos_fc_w, pos_fc_b, pos_ln_g, pos_ln_b):
    up = jnp.repeat(jnp.repeat(encode_src, 2, axis=2), 2, axis=3)
    cat4d = jnp.concatenate([up, feat_4x], axis=1)
    cat = cat4d.reshape(B, 2 * C, HW)

    esu_t, pos_all, small, meta = _dense_call(
        cat, conv_w, conv_b, mem_fc_w, mem_fc_b, mem_ln_g, mem_ln_b,
        cls_w, cls_b, mlp_w1, mlp_b1, mlp_w2, mlp_b2, mlp_w3, mlp_b3,
        pos_fc_w, pos_fc_b, pos_ln_g, pos_ln_b)

    prop_np = _prop_table()
    validv = jnp.asarray((prop_np[:, 0] < 1e5))
    scores = _oracle_scores(cat4d, conv_w, conv_b, mem_fc_w, mem_fc_b,
                            mem_ln_g, mem_ln_b, cls_w, cls_b,
                            ~validv[None, :, None])

    s = scores
    iota = jnp.arange(HW, dtype=jnp.int32)
    gt = (s[:, None, :] > s[:, :, None]).sum(-1, dtype=jnp.int32)
    eq = (s[:, None, :] == s[:, :, None]) & (iota[None, None, :] < iota[None, :, None])
    rank = gt + eq.sum(-1, dtype=jnp.int32)
    sel = rank < K

    refx = small[..., 2]
    refy = small[..., 3]
    cls_out = small[..., 0:2]
    coord_out = jnp.stack([refy, refx], axis=-1)

    bidx = jnp.arange(B)[:, None]
    rr = jnp.where(sel, rank, HW)
    ref_all = jnp.stack([refx, refy], axis=-1)
    reference_points = jnp.zeros((B, HW + 1, 2), jnp.float32).at[bidx, rr].set(ref_all)[:, :K]

    rp_x = jnp.round(refx * W).astype(jnp.int32)
    rp_y = jnp.round(refy * H).astype(jnp.int32)
    pos_idx = jnp.clip(rp_y * W + rp_x, 0, HW - 1)
    packed = jnp.where(sel, (rank << 14) | iota[None, :], -1)
    win = jnp.full((B, HW), -1, jnp.int32).at[bidx, pos_idx].max(packed)
    has = win >= 0
    wtok = jnp.where(has, win & (HW - 1), 0)

    wmeta = jnp.take_along_axis(meta, wtok[..., None], axis=1)
    nbr = wmeta[..., 0:4].astype(jnp.int32)
    wgt = wmeta[..., 4:8] * has[..., None].astype(jnp.float32)
    qf = jnp.zeros((B, HW, C), jnp.float32)
    for t in range(4):
        rows = jnp.take_along_axis(esu_t, nbr[..., t][..., None], axis=1)
        qf = qf + rows * wgt[..., t][..., None]
    qpf = jnp.take_along_axis(pos_all, wtok[..., None], axis=1)
    qpf = qpf * has[..., None].astype(jnp.float32)

    query = jnp.transpose(qf, (0, 2, 1)).reshape(B, C, H, W)
    query_pos = jnp.transpose(qpf, (1, 0, 2))
    return (query, query_pos, reference_points, cls_out, coord_out)


# speedup vs baseline: 1.1090x; 1.1090x over previous
import functools
import math

import jax
import jax.numpy as jnp
import numpy as np
from jax import lax
from jax.experimental import pallas as pl
from jax.experimental.pallas import tpu as pltpu, tpu_sc as plsc

B, C, H, W = 4, 256, 128, 128
HW = H * W
N = B * HW
K = int(0.9 * HW)
TT = 1024
NT = HW // TT
CH = 32
LOG1E4 = math.log(10000.0)


def _dense_body(cat_ref, cw_ref, cb_ref, mfwT_ref, mfb_ref, mlg_ref, mlb_ref,
                clswT_ref, clsb_ref, w1T_ref, b1_ref, w2T_ref, b2_ref,
                w3T_ref, b3_ref, prop_ref,
                esu_ref, small_ref, meta_ref):
    t = pl.program_id(1)
    cat = cat_ref[0]
    esu_cm = jnp.dot(cw_ref[...], cat, preferred_element_type=jnp.float32)
    esu_cm = esu_cm + cb_ref[...]
    esu_t = esu_cm.T
    esu_ref[0] = esu_t

    ii = jax.lax.broadcasted_iota(jnp.int32, (TT, 1), 0)
    tok = t * TT + ii
    x = tok & (W - 1)
    y = tok >> 7
    valid = (x >= 1) & (x <= W - 2) & (y >= 1) & (y <= H - 2)
    om = jnp.where(valid, esu_t, 0.0)

    om1 = jnp.dot(om, mfwT_ref[...], preferred_element_type=jnp.float32) + mfb_ref[...]
    m = jnp.mean(om1, axis=1, keepdims=True)
    v = jnp.mean((om1 - m) ** 2, axis=1, keepdims=True)
    ln = (om1 - m) / jnp.sqrt(v + 1e-5) * mlg_ref[...] + mlb_ref[...]

    clsp = jnp.dot(ln, clswT_ref[...], preferred_element_type=jnp.float32) + clsb_ref[...]
    h1 = jnp.maximum(jnp.dot(ln, w1T_ref[...], preferred_element_type=jnp.float32) + b1_ref[...], 0.0)
    h2 = jnp.maximum(jnp.dot(h1, w2T_ref[...], preferred_element_type=jnp.float32) + b2_ref[...], 0.0)
    dpad = jnp.dot(h2, w3T_ref[...], preferred_element_type=jnp.float32) + b3_ref[...]

    prop = prop_ref[...]
    ux = dpad[:, 0:1] + prop[:, 0:1]
    uy = dpad[:, 1:2] + prop[:, 1:2]
    refx = jax.nn.sigmoid(ux)
    refy = jax.nn.sigmoid(uy)

    gx = refx * W - 0.5
    gy = refy * H - 0.5
    x0 = jnp.floor(gx)
    y0 = jnp.floor(gy)
    fx = gx - x0
    fy = gy - y0

    def nb(xi, yi):
        ok = (xi >= 0) & (xi < W) & (yi >= 0) & (yi < H)
        idx = jnp.clip(yi, 0, H - 1) * W + jnp.clip(xi, 0, W - 1)
        return idx, ok.astype(jnp.float32)

    i00, v00 = nb(x0, y0)
    i01, v01 = nb(x0 + 1, y0)
    i10, v10 = nb(x0, y0 + 1)
    i11, v11 = nb(x0 + 1, y0 + 1)
    w00 = (1 - fx) * (1 - fy) * v00
    w01 = fx * (1 - fy) * v01
    w10 = (1 - fx) * fy * v10
    w11 = fx * fy * v11
    z = jnp.zeros_like(ux)
    meta_ref[0] = jnp.concatenate(
        [i00, i01, i10, i11, w00, w01, w10, w11, ux, uy, z, z, z, z, z, z], axis=1)
    small_ref[0] = jnp.concatenate(
        [clsp[:, 0:1], clsp[:, 1:2], refx, refy, ux, uy, ux, uy], axis=1)


def _prop_table():
    gy, gx = np.meshgrid(np.arange(H, dtype=np.float64), np.arange(W, dtype=np.float64), indexing='ij')
    px = ((gx + 0.5) / W).astype(np.float32)
    py = ((gy + 0.5) / H).astype(np.float32)
    lx = np.log(px / (1.0 - px)).astype(np.float32)
    ly = np.log(py / (1.0 - py)).astype(np.float32)
    valid = (px > 0.01) & (px < 0.99) & (py > 0.01) & (py < 0.99)
    lx = np.where(valid, lx, 1e6).astype(np.float32)
    ly = np.where(valid, ly, 1e6).astype(np.float32)
    z = np.zeros_like(lx)
    return np.stack([lx, ly, z, z, z, z, z, z], axis=-1).reshape(HW, 8)


def _dense_call(cat, conv_w, conv_b, mem_fc_w, mem_fc_b, mem_ln_g, mem_ln_b,
                cls_w, cls_b, mlp_w1, mlp_b1, mlp_w2, mlp_b2, mlp_w3, mlp_b3):
    clswT = jnp.zeros((C, 128), jnp.float32).at[:, :2].set(cls_w.T)
    clsbp = jnp.zeros((1, 128), jnp.float32).at[:, :2].set(cls_b)
    w3T = jnp.zeros((C, 128), jnp.float32).at[:, :2].set(mlp_w3.T)
    b3p = jnp.zeros((1, 128), jnp.float32).at[:, :2].set(mlp_b3)
    prop = jnp.asarray(_prop_table())

    full = lambda *shape: pl.BlockSpec(shape, lambda b, t: (0,) * len(shape))
    return pl.pallas_call(
        _dense_body,
        grid=(B, NT),
        in_specs=[
            pl.BlockSpec((1, 2 * C, TT), lambda b, t: (b, 0, t)),
            full(C, 2 * C), full(C, 1),
            full(C, C), full(1, C), full(1, C), full(1, C),
            full(C, 128), full(1, 128),
            full(C, C), full(1, C), full(C, C), full(1, C),
            full(C, 128), full(1, 128),
            pl.BlockSpec((TT, 8), lambda b, t: (t, 0)),
        ],
        out_specs=[
            pl.BlockSpec((1, TT, C), lambda b, t: (b, t, 0)),
            pl.BlockSpec((1, TT, 8), lambda b, t: (b, t, 0)),
            pl.BlockSpec((1, TT, 16), lambda b, t: (b, t, 0)),
        ],
        out_shape=[
            jax.ShapeDtypeStruct((B, HW, C), jnp.float32),
            jax.ShapeDtypeStruct((B, HW, 8), jnp.float32),
            jax.ShapeDtypeStruct((B, HW, 16), jnp.float32),
        ],
    )(cat, conv_w, conv_b.reshape(C, 1),
      mem_fc_w.T, mem_fc_b.reshape(1, C), mem_ln_g.reshape(1, C), mem_ln_b.reshape(1, C),
      clswT, clsbp,
      mlp_w1.T, mlp_b1.reshape(1, C), mlp_w2.T, mlp_b2.reshape(1, C),
      w3T, b3p, prop)


def _make_placement():
    info = plsc.get_sparse_core_info()
    NC, NS = info.num_cores, info.num_subcores
    NW = NC * NS
    per_w = N // NW
    n_chunks = per_w // CH
    mesh = plsc.VectorSubcoreMesh(core_axis_name="c", subcore_axis_name="s")

    @functools.partial(
        pl.kernel, mesh=mesh,
        out_type=jax.ShapeDtypeStruct((N, C), jnp.float32),
        scratch_types=[
            pltpu.VMEM((4 * CH,), jnp.int32),
            pltpu.VMEM((CH, 16), jnp.float32),
            pltpu.VMEM((4 * CH, C), jnp.float32),
            pltpu.VMEM((CH, C), jnp.float32),
            pltpu.SemaphoreType.DMA,
        ],
    )
    def k(esu_hbm, wmeta_hbm, nbr_hbm, qf_hbm,
          nidx_v, meta_v, erows_v, out_v, sem):
        wid = lax.axis_index("s") * NC + lax.axis_index("c")
        wbase = wid * per_w

        def chunk_body(ci, carry):
            base = wbase + ci * CH
            pltpu.sync_copy(nbr_hbm.at[pl.ds(4 * base, 4 * CH)], nidx_v)
            pltpu.sync_copy(wmeta_hbm.at[pl.ds(base, CH)], meta_v)
            pltpu.async_copy(esu_hbm.at[nidx_v], erows_v, sem).wait()
            for s in range(CH):
                mrow = meta_v[s, pl.ds(0, 16)]
                w0 = mrow[4]
                w1 = mrow[5]
                w2 = mrow[6]
                w3 = mrow[7]
                for cv in range(C // 16):
                    sl = pl.ds(16 * cv, 16)
                    out_v[s, sl] = (w0 * erows_v[0 * CH + s, sl]
                                    + w1 * erows_v[1 * CH + s, sl]
                                    + w2 * erows_v[2 * CH + s, sl]
                                    + w3 * erows_v[3 * CH + s, sl])
            pltpu.sync_copy(out_v, qf_hbm.at[pl.ds(base, CH)])
            return carry

        lax.fori_loop(0, n_chunks, chunk_body, 0, unroll=False)

    return k


def _pos_body(wuxy_ref, win_ref, pfwT_ref, pfb_ref, plg_ref, plb_ref, qpf_ref):
    ux = wuxy_ref[:, 0:1]
    uy = wuxy_ref[:, 1:2]
    has = (win_ref[...] >= 0).astype(jnp.float32)
    refx = jax.nn.sigmoid(ux)
    refy = jax.nn.sigmoid(uy)
    k = jax.lax.broadcasted_iota(jnp.int32, (1, C), 1)
    j = (k & 127) >> 1
    tdiv = jnp.exp((2.0 * j.astype(jnp.float32) / 128.0) * LOG1E4)
    pxy = jnp.where(k < 128, refx * (2.0 * math.pi), refy * (2.0 * math.pi))
    ang = pxy / tdiv
    even = (k & 1) == 0
    posf = jnp.where(even, jnp.sin(ang), jnp.cos(ang))
    pn = jnp.dot(posf, pfwT_ref[...], preferred_element_type=jnp.float32) + pfb_ref[...]
    m2 = jnp.mean(pn, axis=1, keepdims=True)
    v2 = jnp.mean((pn - m2) ** 2, axis=1, keepdims=True)
    qpf_ref[...] = ((pn - m2) / jnp.sqrt(v2 + 1e-5) * plg_ref[...] + plb_ref[...]) * has


def _pos_call(wuxy, win_col, pos_fc_w, pos_fc_b, pos_ln_g, pos_ln_b):
    full = lambda *shape: pl.BlockSpec(shape, lambda t: (0,) * len(shape))
    return pl.pallas_call(
        _pos_body,
        grid=(N // TT,),
        in_specs=[
            pl.BlockSpec((TT, 2), lambda t: (t, 0)),
            pl.BlockSpec((TT, 1), lambda t: (t, 0)),
            full(C, C), full(1, C), full(1, C), full(1, C),
        ],
        out_specs=pl.BlockSpec((TT, C), lambda t: (t, 0)),
        out_shape=jax.ShapeDtypeStruct((N, C), jnp.float32),
    )(wuxy, win_col, pos_fc_w.T, pos_fc_b.reshape(1, C),
      pos_ln_g.reshape(1, C), pos_ln_b.reshape(1, C))


def _oracle_scores(cat4d, conv_w, conv_b, mem_fc_w, mem_fc_b, mem_ln_g, mem_ln_b,
                   cls_w, cls_b, invalid):
    esu = jnp.einsum('bchw,oc->bohw', cat4d, conv_w) + conv_b[None, :, None, None]
    output_memory = jnp.transpose(esu.reshape(B, C, HW), (0, 2, 1))
    output_memory = jnp.where(invalid, 0.0, output_memory)
    om = output_memory @ mem_fc_w.T + mem_fc_b
    m = jnp.mean(om, axis=-1, keepdims=True)
    v = jnp.var(om, axis=-1, keepdims=True)
    om = (om - m) / jnp.sqrt(v + 1e-5) * mem_ln_g + mem_ln_b
    cls = om @ cls_w.T + cls_b
    return jax.nn.softmax(cls, axis=-1)[..., 1]


def kernel(encode_src, feat_4x, mask, conv_w, conv_b, mem_fc_w, mem_fc_b,
           mem_ln_g, mem_ln_b, cls_w, cls_b, mlp_w1, mlp_b1, mlp_w2, mlp_b2,
           mlp_w3, mlp_b3, pos_fc_w, pos_fc_b, pos_ln_g, pos_ln_b):
    up = jnp.repeat(jnp.repeat(encode_src, 2, axis=2), 2, axis=3)
    cat4d = jnp.concatenate([up, feat_4x], axis=1)
    cat = cat4d.reshape(B, 2 * C, HW)

    esu_t, small, meta = _dense_call(
        cat, conv_w, conv_b, mem_fc_w, mem_fc_b, mem_ln_g, mem_ln_b,
        cls_w, cls_b, mlp_w1, mlp_b1, mlp_w2, mlp_b2, mlp_w3, mlp_b3)

    prop_np = _prop_table()
    validv = jnp.asarray(prop_np[:, 0] < 1e5)
    scores = _oracle_scores(cat4d, conv_w, conv_b, mem_fc_w, mem_fc_b,
                            mem_ln_g, mem_ln_b, cls_w, cls_b,
                            ~validv[None, :, None])

    s = scores
    iota = jnp.arange(HW, dtype=jnp.int32)
    gt = (s[:, None, :] > s[:, :, None]).sum(-1, dtype=jnp.int32)
    eq = (s[:, None, :] == s[:, :, None]) & (iota[None, None, :] < iota[None, :, None])
    rank = gt + eq.sum(-1, dtype=jnp.int32)
    sel = rank < K

    refx = small[..., 2]
    refy = small[..., 3]
    cls_out = small[..., 0:2]
    coord_out = jnp.stack([refy, refx], axis=-1)

    bidx = jnp.arange(B)[:, None]
    rr = jnp.where(sel, rank, HW)
    ref_all = jnp.stack([refx, refy], axis=-1)
    reference_points = jnp.zeros((B, HW + 1, 2), jnp.float32).at[bidx, rr].set(ref_all)[:, :K]

    rp_x = jnp.round(refx * W).astype(jnp.int32)
    rp_y = jnp.round(refy * H).astype(jnp.int32)
    pos_idx = jnp.clip(rp_y * W + rp_x, 0, HW - 1)
    packed = jnp.where(sel, (rank << 14) | iota[None, :], -1)
    win = jnp.full((B, HW), -1, jnp.int32).at[bidx, pos_idx].max(packed)
    has = win >= 0
    wtok = jnp.where(has, win & (HW - 1), 0)

    wmeta = jnp.take_along_axis(meta, wtok[..., None], axis=1)
    hasf = has[..., None].astype(jnp.float32)
    boff = (jnp.arange(B, dtype=jnp.float32) * HW)[:, None, None]
    wmeta = jnp.concatenate(
        [wmeta[..., 0:4] + boff, wmeta[..., 4:8] * hasf, wmeta[..., 8:16]], axis=-1)
    nbr = wmeta[..., 0:4].astype(jnp.int32)
    nbrflat = jnp.transpose(nbr.reshape(N // CH, CH, 4), (0, 2, 1)).reshape(4 * N)

    place = _make_placement()
    qf = place(esu_t.reshape(N, C), wmeta.reshape(N, 16), nbrflat)

    wuxy = wmeta[..., 8:10].reshape(N, 2)
    qpf = _pos_call(wuxy, win.reshape(N, 1), pos_fc_w, pos_fc_b, pos_ln_g, pos_ln_b)

    query = jnp.transpose(qf.reshape(B, HW, C), (0, 2, 1)).reshape(B, C, H, W)
    query_pos = jnp.transpose(qpf.reshape(B, HW, C), (1, 0, 2))
    return (query, query_pos, reference_points, cls_out, coord_out)


# speedup vs baseline: 1.1881x; 1.0713x over previous
import functools
import math

import jax
import jax.numpy as jnp
import numpy as np
from jax import lax
from jax.experimental import pallas as pl
from jax.experimental.pallas import tpu as pltpu, tpu_sc as plsc

B, C, H, W = 4, 256, 128, 128
HW = H * W
N = B * HW
K = int(0.9 * HW)
TT = 1024
NT = HW // TT
CH = 16
LOG1E4 = math.log(10000.0)


def _dense_body(cat_ref, cw_ref, cb_ref, mfwT_ref, mfb_ref, mlg_ref, mlb_ref,
                clswT_ref, clsb_ref, w1T_ref, b1_ref, w2T_ref, b2_ref,
                w3T_ref, b3_ref, prop_ref,
                esu_ref, small_ref, meta_ref):
    t = pl.program_id(1)
    cat = cat_ref[0]
    esu_cm = jnp.dot(cw_ref[...], cat, preferred_element_type=jnp.float32)
    esu_cm = esu_cm + cb_ref[...]
    esu_t = esu_cm.T
    esu_ref[0] = esu_t

    ii = jax.lax.broadcasted_iota(jnp.int32, (TT, 1), 0)
    tok = t * TT + ii
    x = tok & (W - 1)
    y = tok >> 7
    valid = (x >= 1) & (x <= W - 2) & (y >= 1) & (y <= H - 2)
    om = jnp.where(valid, esu_t, 0.0)

    om1 = jnp.dot(om, mfwT_ref[...], preferred_element_type=jnp.float32) + mfb_ref[...]
    m = jnp.mean(om1, axis=1, keepdims=True)
    v = jnp.mean((om1 - m) ** 2, axis=1, keepdims=True)
    ln = (om1 - m) / jnp.sqrt(v + 1e-5) * mlg_ref[...] + mlb_ref[...]

    clsp = jnp.dot(ln, clswT_ref[...], preferred_element_type=jnp.float32) + clsb_ref[...]
    h1 = jnp.maximum(jnp.dot(ln, w1T_ref[...], preferred_element_type=jnp.float32) + b1_ref[...], 0.0)
    h2 = jnp.maximum(jnp.dot(h1, w2T_ref[...], preferred_element_type=jnp.float32) + b2_ref[...], 0.0)
    dpad = jnp.dot(h2, w3T_ref[...], preferred_element_type=jnp.float32) + b3_ref[...]

    prop = prop_ref[...]
    ux = dpad[:, 0:1] + prop[:, 0:1]
    uy = dpad[:, 1:2] + prop[:, 1:2]
    refx = jax.nn.sigmoid(ux)
    refy = jax.nn.sigmoid(uy)

    gx = refx * W - 0.5
    gy = refy * H - 0.5
    x0 = jnp.floor(gx)
    y0 = jnp.floor(gy)
    fx = gx - x0
    fy = gy - y0

    def nb(xi, yi):
        ok = (xi >= 0) & (xi < W) & (yi >= 0) & (yi < H)
        idx = jnp.clip(yi, 0, H - 1) * W + jnp.clip(xi, 0, W - 1)
        return idx, ok.astype(jnp.float32)

    i00, v00 = nb(x0, y0)
    i01, v01 = nb(x0 + 1, y0)
    i10, v10 = nb(x0, y0 + 1)
    i11, v11 = nb(x0 + 1, y0 + 1)
    w00 = (1 - fx) * (1 - fy) * v00
    w01 = fx * (1 - fy) * v01
    w10 = (1 - fx) * fy * v10
    w11 = fx * fy * v11
    z = jnp.zeros_like(ux)
    meta_ref[0] = jnp.concatenate(
        [i00, i01, i10, i11, w00, w01, w10, w11, ux, uy, z, z, z, z, z, z], axis=1)
    small_ref[0] = jnp.concatenate(
        [clsp[:, 0:1], clsp[:, 1:2], refx, refy, ux, uy, ux, uy], axis=1)


def _prop_table():
    gy, gx = np.meshgrid(np.arange(H, dtype=np.float64), np.arange(W, dtype=np.float64), indexing='ij')
    px = ((gx + 0.5) / W).astype(np.float32)
    py = ((gy + 0.5) / H).astype(np.float32)
    lx = np.log(px / (1.0 - px)).astype(np.float32)
    ly = np.log(py / (1.0 - py)).astype(np.float32)
    valid = (px > 0.01) & (px < 0.99) & (py > 0.01) & (py < 0.99)
    lx = np.where(valid, lx, 1e6).astype(np.float32)
    ly = np.where(valid, ly, 1e6).astype(np.float32)
    z = np.zeros_like(lx)
    return np.stack([lx, ly, z, z, z, z, z, z], axis=-1).reshape(HW, 8)


def _dense_call(cat, conv_w, conv_b, mem_fc_w, mem_fc_b, mem_ln_g, mem_ln_b,
                cls_w, cls_b, mlp_w1, mlp_b1, mlp_w2, mlp_b2, mlp_w3, mlp_b3):
    clswT = jnp.zeros((C, 128), jnp.float32).at[:, :2].set(cls_w.T)
    clsbp = jnp.zeros((1, 128), jnp.float32).at[:, :2].set(cls_b)
    w3T = jnp.zeros((C, 128), jnp.float32).at[:, :2].set(mlp_w3.T)
    b3p = jnp.zeros((1, 128), jnp.float32).at[:, :2].set(mlp_b3)
    prop = jnp.asarray(_prop_table())

    full = lambda *shape: pl.BlockSpec(shape, lambda b, t: (0,) * len(shape))
    return pl.pallas_call(
        _dense_body,
        grid=(B, NT),
        in_specs=[
            pl.BlockSpec((1, 2 * C, TT), lambda b, t: (b, 0, t)),
            full(C, 2 * C), full(C, 1),
            full(C, C), full(1, C), full(1, C), full(1, C),
            full(C, 128), full(1, 128),
            full(C, C), full(1, C), full(C, C), full(1, C),
            full(C, 128), full(1, 128),
            pl.BlockSpec((TT, 8), lambda b, t: (t, 0)),
        ],
        out_specs=[
            pl.BlockSpec((1, TT, C), lambda b, t: (b, t, 0)),
            pl.BlockSpec((1, TT, 8), lambda b, t: (b, t, 0)),
            pl.BlockSpec((1, TT, 16), lambda b, t: (b, t, 0)),
        ],
        out_shape=[
            jax.ShapeDtypeStruct((B, HW, C), jnp.float32),
            jax.ShapeDtypeStruct((B, HW, 8), jnp.float32),
            jax.ShapeDtypeStruct((B, HW, 16), jnp.float32),
        ],
    )(cat, conv_w, conv_b.reshape(C, 1),
      mem_fc_w.T, mem_fc_b.reshape(1, C), mem_ln_g.reshape(1, C), mem_ln_b.reshape(1, C),
      clswT, clsbp,
      mlp_w1.T, mlp_b1.reshape(1, C), mlp_w2.T, mlp_b2.reshape(1, C),
      w3T, b3p, prop)


def _make_placement():
    info = plsc.get_sparse_core_info()
    NC, NS = info.num_cores, info.num_subcores
    NW = NC * NS
    per_w = N // NW
    n_chunks = per_w // CH
    mesh = plsc.VectorSubcoreMesh(core_axis_name="c", subcore_axis_name="s")

    @functools.partial(
        pl.kernel, mesh=mesh,
        out_type=jax.ShapeDtypeStruct((N, C), jnp.float32),
        scratch_types=[
            pltpu.VMEM((4 * per_w,), jnp.int32),
            pltpu.VMEM((CH, 16), jnp.float32),
            pltpu.VMEM((CH, 16), jnp.float32),
            pltpu.VMEM((4 * CH, C), jnp.float32),
            pltpu.VMEM((4 * CH, C), jnp.float32),
            pltpu.VMEM((CH, C), jnp.float32),
            pltpu.VMEM((CH, C), jnp.float32),
            pltpu.SemaphoreType.DMA,
            pltpu.SemaphoreType.DMA,
            pltpu.SemaphoreType.DMA,
            pltpu.SemaphoreType.DMA,
            pltpu.SemaphoreType.DMA,
            pltpu.SemaphoreType.DMA,
        ],
    )
    def k(esu_hbm, wmeta_hbm, nbr_hbm, qf_hbm,
          nidx_v, mt0_v, mt1_v, er0_v, er1_v, out0_v, out1_v,
          g0, g1, m0, m1, o0, o1):
        wid = lax.axis_index("s") * NC + lax.axis_index("c")
        wbase = wid * per_w
        metas = (mt0_v, mt1_v)
        erows = (er0_v, er1_v)
        outs = (out0_v, out1_v)
        gsems = (g0, g1)
        msems = (m0, m1)
        osems = (o0, o1)

        pltpu.sync_copy(nbr_hbm.at[pl.ds(4 * wbase, 4 * per_w)], nidx_v)

        def gather(ci, buf):
            idx = nidx_v.at[pl.ds(4 * CH * ci, 4 * CH)]
            pltpu.async_copy(esu_hbm.at[idx], erows[buf], gsems[buf])
            pltpu.async_copy(wmeta_hbm.at[pl.ds(wbase + CH * ci, CH)],
                             metas[buf], msems[buf])

        gather(0, 0)
        gather(1, 1)
        n_pairs = n_chunks // 2

        def do_chunk(p, buf, off):
            ci = 2 * p + off
            pltpu.make_async_copy(esu_hbm.at[nidx_v.at[pl.ds(0, 4 * CH)]],
                                  erows[buf], gsems[buf]).wait()
            pltpu.make_async_copy(wmeta_hbm.at[pl.ds(0, CH)],
                                  metas[buf], msems[buf]).wait()

            @pl.when(p >= 1)
            def _():
                pltpu.make_async_copy(outs[buf], qf_hbm.at[pl.ds(0, CH)],
                                      osems[buf]).wait()

            er = erows[buf]
            meta_v = metas[buf]
            out_v = outs[buf]
            for s in range(CH):
                mrow = meta_v[s, pl.ds(0, 16)]
                w0 = mrow[4]
                w1 = mrow[5]
                w2 = mrow[6]
                w3 = mrow[7]
                for cv in range(C // 16):
                    sl = pl.ds(16 * cv, 16)
                    out_v[s, sl] = (w0 * er[0 * CH + s, sl]
                                    + w1 * er[1 * CH + s, sl]
                                    + w2 * er[2 * CH + s, sl]
                                    + w3 * er[3 * CH + s, sl])
            pltpu.async_copy(out_v, qf_hbm.at[pl.ds(wbase + CH * ci, CH)], osems[buf])

            @pl.when(ci + 2 < n_chunks)
            def _():
                gather(ci + 2, buf)

        def pair_body(p, carry):
            do_chunk(p, 0, 0)
            do_chunk(p, 1, 1)
            return carry

        lax.fori_loop(0, n_pairs, pair_body, 0, unroll=False)
        pltpu.make_async_copy(outs[0], qf_hbm.at[pl.ds(0, CH)], osems[0]).wait()
        pltpu.make_async_copy(outs[1], qf_hbm.at[pl.ds(0, CH)], osems[1]).wait()

    return k


def _pos_body(wuxy_ref, win_ref, pfwT_ref, pfb_ref, plg_ref, plb_ref, qpf_ref):
    ux = wuxy_ref[:, 0:1]
    uy = wuxy_ref[:, 1:2]
    has = (win_ref[...] >= 0).astype(jnp.float32)
    refx = jax.nn.sigmoid(ux)
    refy = jax.nn.sigmoid(uy)
    k = jax.lax.broadcasted_iota(jnp.int32, (1, C), 1)
    j = (k & 127) >> 1
    tdiv = jnp.exp((2.0 * j.astype(jnp.float32) / 128.0) * LOG1E4)
    pxy = jnp.where(k < 128, refx * (2.0 * math.pi), refy * (2.0 * math.pi))
    ang = pxy / tdiv
    even = (k & 1) == 0
    posf = jnp.where(even, jnp.sin(ang), jnp.cos(ang))
    pn = jnp.dot(posf, pfwT_ref[...], preferred_element_type=jnp.float32) + pfb_ref[...]
    m2 = jnp.mean(pn, axis=1, keepdims=True)
    v2 = jnp.mean((pn - m2) ** 2, axis=1, keepdims=True)
    qpf_ref[...] = ((pn - m2) / jnp.sqrt(v2 + 1e-5) * plg_ref[...] + plb_ref[...]) * has


def _pos_call(wuxy, win_col, pos_fc_w, pos_fc_b, pos_ln_g, pos_ln_b):
    full = lambda *shape: pl.BlockSpec(shape, lambda t: (0,) * len(shape))
    return pl.pallas_call(
        _pos_body,
        grid=(N // TT,),
        in_specs=[
            pl.BlockSpec((TT, 2), lambda t: (t, 0)),
            pl.BlockSpec((TT, 1), lambda t: (t, 0)),
            full(C, C), full(1, C), full(1, C), full(1, C),
        ],
        out_specs=pl.BlockSpec((TT, C), lambda t: (t, 0)),
        out_shape=jax.ShapeDtypeStruct((N, C), jnp.float32),
    )(wuxy, win_col, pos_fc_w.T, pos_fc_b.reshape(1, C),
      pos_ln_g.reshape(1, C), pos_ln_b.reshape(1, C))


def _oracle_scores(cat4d, conv_w, conv_b, mem_fc_w, mem_fc_b, mem_ln_g, mem_ln_b,
                   cls_w, cls_b, invalid):
    esu = jnp.einsum('bchw,oc->bohw', cat4d, conv_w) + conv_b[None, :, None, None]
    output_memory = jnp.transpose(esu.reshape(B, C, HW), (0, 2, 1))
    output_memory = jnp.where(invalid, 0.0, output_memory)
    om = output_memory @ mem_fc_w.T + mem_fc_b
    m = jnp.mean(om, axis=-1, keepdims=True)
    v = jnp.var(om, axis=-1, keepdims=True)
    om = (om - m) / jnp.sqrt(v + 1e-5) * mem_ln_g + mem_ln_b
    cls = om @ cls_w.T + cls_b
    return jax.nn.softmax(cls, axis=-1)[..., 1]


def kernel(encode_src, feat_4x, mask, conv_w, conv_b, mem_fc_w, mem_fc_b,
           mem_ln_g, mem_ln_b, cls_w, cls_b, mlp_w1, mlp_b1, mlp_w2, mlp_b2,
           mlp_w3, mlp_b3, pos_fc_w, pos_fc_b, pos_ln_g, pos_ln_b):
    up = jnp.repeat(jnp.repeat(encode_src, 2, axis=2), 2, axis=3)
    cat4d = jnp.concatenate([up, feat_4x], axis=1)
    cat = cat4d.reshape(B, 2 * C, HW)

    esu_t, small, meta = _dense_call(
        cat, conv_w, conv_b, mem_fc_w, mem_fc_b, mem_ln_g, mem_ln_b,
        cls_w, cls_b, mlp_w1, mlp_b1, mlp_w2, mlp_b2, mlp_w3, mlp_b3)

    prop_np = _prop_table()
    validv = jnp.asarray(prop_np[:, 0] < 1e5)
    scores = _oracle_scores(cat4d, conv_w, conv_b, mem_fc_w, mem_fc_b,
                            mem_ln_g, mem_ln_b, cls_w, cls_b,
                            ~validv[None, :, None])

    s = scores
    iota = jnp.arange(HW, dtype=jnp.int32)
    gt = (s[:, None, :] > s[:, :, None]).sum(-1, dtype=jnp.int32)
    eq = (s[:, None, :] == s[:, :, None]) & (iota[None, None, :] < iota[None, :, None])
    rank = gt + eq.sum(-1, dtype=jnp.int32)
    sel = rank < K

    refx = small[..., 2]
    refy = small[..., 3]
    cls_out = small[..., 0:2]
    coord_out = jnp.stack([refy, refx], axis=-1)

    bidx = jnp.arange(B)[:, None]
    rr = jnp.where(sel, rank, HW)
    ref_all = jnp.stack([refx, refy], axis=-1)
    reference_points = jnp.zeros((B, HW + 1, 2), jnp.float32).at[bidx, rr].set(ref_all)[:, :K]

    rp_x = jnp.round(refx * W).astype(jnp.int32)
    rp_y = jnp.round(refy * H).astype(jnp.int32)
    pos_idx = jnp.clip(rp_y * W + rp_x, 0, HW - 1)
    packed = jnp.where(sel, (rank << 14) | iota[None, :], -1)
    win = jnp.full((B, HW), -1, jnp.int32).at[bidx, pos_idx].max(packed)
    has = win >= 0
    wtok = jnp.where(has, win & (HW - 1), 0)

    wmeta = jnp.take_along_axis(meta, wtok[..., None], axis=1)
    hasf = has[..., None].astype(jnp.float32)
    boff = (jnp.arange(B, dtype=jnp.float32) * HW)[:, None, None]
    wmeta = jnp.concatenate(
        [wmeta[..., 0:4] + boff, wmeta[..., 4:8] * hasf, wmeta[..., 8:16]], axis=-1)
    nbr = wmeta[..., 0:4].astype(jnp.int32)
    nbrflat = jnp.transpose(nbr.reshape(N // CH, CH, 4), (0, 2, 1)).reshape(4 * N)

    place = _make_placement()
    qf = place(esu_t.reshape(N, C), wmeta.reshape(N, 16), nbrflat)

    wuxy = wmeta[..., 8:10].reshape(N, 2)
    qpf = _pos_call(wuxy, win.reshape(N, 1), pos_fc_w, pos_fc_b, pos_ln_g, pos_ln_b)

    query = jnp.transpose(qf.reshape(B, HW, C), (0, 2, 1)).reshape(B, C, H, W)
    query_pos = jnp.transpose(qpf.reshape(B, HW, C), (1, 0, 2))
    return (query, query_pos, reference_points, cls_out, coord_out)


# speedup vs baseline: 1.5924x; 1.3403x over previous
import functools
import math

import jax
import jax.numpy as jnp
import numpy as np
from jax import lax
from jax.experimental import pallas as pl
from jax.experimental.pallas import tpu as pltpu, tpu_sc as plsc

B, C, H, W = 4, 256, 128, 128
HW = H * W
N = B * HW
K = int(0.9 * HW)
TT = 1024
NT = HW // TT
CH = 16
LOG1E4 = math.log(10000.0)


def _dense_body(cat_ref, cw_ref, cb_ref, mfwT_ref, mfb_ref, mlg_ref, mlb_ref,
                clswT_ref, clsb_ref, w1T_ref, b1_ref, w2T_ref, b2_ref,
                w3T_ref, b3_ref, prop_ref,
                esu_ref, small_ref, meta_ref):
    t = pl.program_id(1)
    cat = cat_ref[0]
    esu_cm = jnp.dot(cw_ref[...], cat, preferred_element_type=jnp.float32)
    esu_cm = esu_cm + cb_ref[...]
    esu_t = esu_cm.T
    esu_ref[0] = esu_t

    ii = jax.lax.broadcasted_iota(jnp.int32, (TT, 1), 0)
    tok = t * TT + ii
    x = tok & (W - 1)
    y = tok >> 7
    valid = (x >= 1) & (x <= W - 2) & (y >= 1) & (y <= H - 2)
    om = jnp.where(valid, esu_t, 0.0)

    om1 = jnp.dot(om, mfwT_ref[...], preferred_element_type=jnp.float32) + mfb_ref[...]
    m = jnp.mean(om1, axis=1, keepdims=True)
    v = jnp.mean((om1 - m) ** 2, axis=1, keepdims=True)
    ln = (om1 - m) / jnp.sqrt(v + 1e-5) * mlg_ref[...] + mlb_ref[...]

    clsp = jnp.dot(ln, clswT_ref[...], preferred_element_type=jnp.float32) + clsb_ref[...]
    h1 = jnp.maximum(jnp.dot(ln, w1T_ref[...], preferred_element_type=jnp.float32) + b1_ref[...], 0.0)
    h2 = jnp.maximum(jnp.dot(h1, w2T_ref[...], preferred_element_type=jnp.float32) + b2_ref[...], 0.0)
    dpad = jnp.dot(h2, w3T_ref[...], preferred_element_type=jnp.float32) + b3_ref[...]

    prop = prop_ref[...]
    ux = dpad[:, 0:1] + prop[:, 0:1]
    uy = dpad[:, 1:2] + prop[:, 1:2]
    refx = jax.nn.sigmoid(ux)
    refy = jax.nn.sigmoid(uy)

    gx = refx * W - 0.5
    gy = refy * H - 0.5
    x0 = jnp.floor(gx)
    y0 = jnp.floor(gy)
    fx = gx - x0
    fy = gy - y0

    def nb(xi, yi):
        ok = (xi >= 0) & (xi < W) & (yi >= 0) & (yi < H)
        idx = jnp.clip(yi, 0, H - 1) * W + jnp.clip(xi, 0, W - 1)
        return idx, ok.astype(jnp.float32)

    i00, v00 = nb(x0, y0)
    i01, v01 = nb(x0 + 1, y0)
    i10, v10 = nb(x0, y0 + 1)
    i11, v11 = nb(x0 + 1, y0 + 1)
    w00 = (1 - fx) * (1 - fy) * v00
    w01 = fx * (1 - fy) * v01
    w10 = (1 - fx) * fy * v10
    w11 = fx * fy * v11
    z = jnp.zeros_like(ux)
    meta_ref[0] = jnp.concatenate(
        [i00, i01, i10, i11, w00, w01, w10, w11, ux, uy, z, z, z, z, z, z], axis=1)
    small_ref[0] = jnp.concatenate(
        [clsp[:, 0:1], clsp[:, 1:2], refx, refy, ux, uy, ux, uy], axis=1)


def _prop_table():
    gy, gx = np.meshgrid(np.arange(H, dtype=np.float64), np.arange(W, dtype=np.float64), indexing='ij')
    px = ((gx + 0.5) / W).astype(np.float32)
    py = ((gy + 0.5) / H).astype(np.float32)
    lx = np.log(px / (1.0 - px)).astype(np.float32)
    ly = np.log(py / (1.0 - py)).astype(np.float32)
    valid = (px > 0.01) & (px < 0.99) & (py > 0.01) & (py < 0.99)
    lx = np.where(valid, lx, 1e6).astype(np.float32)
    ly = np.where(valid, ly, 1e6).astype(np.float32)
    z = np.zeros_like(lx)
    return np.stack([lx, ly, z, z, z, z, z, z], axis=-1).reshape(HW, 8)


def _dense_call(cat, conv_w, conv_b, mem_fc_w, mem_fc_b, mem_ln_g, mem_ln_b,
                cls_w, cls_b, mlp_w1, mlp_b1, mlp_w2, mlp_b2, mlp_w3, mlp_b3):
    clswT = jnp.zeros((C, 128), jnp.float32).at[:, :2].set(cls_w.T)
    clsbp = jnp.zeros((1, 128), jnp.float32).at[:, :2].set(cls_b)
    w3T = jnp.zeros((C, 128), jnp.float32).at[:, :2].set(mlp_w3.T)
    b3p = jnp.zeros((1, 128), jnp.float32).at[:, :2].set(mlp_b3)
    prop = jnp.asarray(_prop_table())

    full = lambda *shape: pl.BlockSpec(shape, lambda b, t: (0,) * len(shape))
    return pl.pallas_call(
        _dense_body,
        grid=(B, NT),
        in_specs=[
            pl.BlockSpec((1, 2 * C, TT), lambda b, t: (b, 0, t)),
            full(C, 2 * C), full(C, 1),
            full(C, C), full(1, C), full(1, C), full(1, C),
            full(C, 128), full(1, 128),
            full(C, C), full(1, C), full(C, C), full(1, C),
            full(C, 128), full(1, 128),
            pl.BlockSpec((TT, 8), lambda b, t: (t, 0)),
        ],
        out_specs=[
            pl.BlockSpec((1, TT, C), lambda b, t: (b, t, 0)),
            pl.BlockSpec((1, TT, 8), lambda b, t: (b, t, 0)),
            pl.BlockSpec((1, TT, 16), lambda b, t: (b, t, 0)),
        ],
        out_shape=[
            jax.ShapeDtypeStruct((B, HW, C), jnp.float32),
            jax.ShapeDtypeStruct((B, HW, 8), jnp.float32),
            jax.ShapeDtypeStruct((B, HW, 16), jnp.float32),
        ],
    )(cat, conv_w, conv_b.reshape(C, 1),
      mem_fc_w.T, mem_fc_b.reshape(1, C), mem_ln_g.reshape(1, C), mem_ln_b.reshape(1, C),
      clswT, clsbp,
      mlp_w1.T, mlp_b1.reshape(1, C), mlp_w2.T, mlp_b2.reshape(1, C),
      w3T, b3p, prop)


def _make_placement():
    info = plsc.get_sparse_core_info()
    NC, NS = info.num_cores, info.num_subcores
    NW = NC * NS
    per_w = N // NW
    n_chunks = per_w // CH
    mesh = plsc.VectorSubcoreMesh(core_axis_name="c", subcore_axis_name="s")

    @functools.partial(
        pl.kernel, mesh=mesh,
        out_type=jax.ShapeDtypeStruct((N, C), jnp.float32),
        scratch_types=[
            pltpu.VMEM((4 * per_w,), jnp.int32),
            pltpu.VMEM((CH, 16), jnp.float32),
            pltpu.VMEM((CH, 16), jnp.float32),
            pltpu.VMEM((4 * CH, C), jnp.float32),
            pltpu.VMEM((4 * CH, C), jnp.float32),
            pltpu.VMEM((CH, C), jnp.float32),
            pltpu.VMEM((CH, C), jnp.float32),
            pltpu.SemaphoreType.DMA,
            pltpu.SemaphoreType.DMA,
            pltpu.SemaphoreType.DMA,
            pltpu.SemaphoreType.DMA,
            pltpu.SemaphoreType.DMA,
            pltpu.SemaphoreType.DMA,
        ],
    )
    def k(esu_hbm, wmeta_hbm, nbr_hbm, qf_hbm,
          nidx_v, mt0_v, mt1_v, er0_v, er1_v, out0_v, out1_v,
          g0, g1, m0, m1, o0, o1):
        wid = lax.axis_index("s") * NC + lax.axis_index("c")
        wbase = wid * per_w
        metas = (mt0_v, mt1_v)
        erows = (er0_v, er1_v)
        outs = (out0_v, out1_v)
        gsems = (g0, g1)
        msems = (m0, m1)
        osems = (o0, o1)

        pltpu.sync_copy(nbr_hbm.at[pl.ds(4 * wbase, 4 * per_w)], nidx_v)

        def gather(ci, buf):
            idx = nidx_v.at[pl.ds(4 * CH * ci, 4 * CH)]
            pltpu.async_copy(esu_hbm.at[idx], erows[buf], gsems[buf])
            pltpu.async_copy(wmeta_hbm.at[pl.ds(wbase + CH * ci, CH)],
                             metas[buf], msems[buf])

        gather(0, 0)
        gather(1, 1)
        n_pairs = n_chunks // 2

        def do_chunk(p, buf, off):
            ci = 2 * p + off
            pltpu.make_async_copy(esu_hbm.at[nidx_v.at[pl.ds(0, 4 * CH)]],
                                  erows[buf], gsems[buf]).wait()
            pltpu.make_async_copy(wmeta_hbm.at[pl.ds(0, CH)],
                                  metas[buf], msems[buf]).wait()

            @pl.when(p >= 1)
            def _():
                pltpu.make_async_copy(outs[buf], qf_hbm.at[pl.ds(0, CH)],
                                      osems[buf]).wait()

            er = erows[buf]
            meta_v = metas[buf]
            out_v = outs[buf]
            for s in range(CH):
                mrow = meta_v[s, pl.ds(0, 16)]
                w0 = mrow[4]
                w1 = mrow[5]
                w2 = mrow[6]
                w3 = mrow[7]
                for cv in range(C // 16):
                    sl = pl.ds(16 * cv, 16)
                    out_v[s, sl] = (w0 * er[0 * CH + s, sl]
                                    + w1 * er[1 * CH + s, sl]
                                    + w2 * er[2 * CH + s, sl]
                                    + w3 * er[3 * CH + s, sl])
            pltpu.async_copy(out_v, qf_hbm.at[pl.ds(wbase + CH * ci, CH)], osems[buf])

            @pl.when(ci + 2 < n_chunks)
            def _():
                gather(ci + 2, buf)

        def pair_body(p, carry):
            do_chunk(p, 0, 0)
            do_chunk(p, 1, 1)
            return carry

        lax.fori_loop(0, n_pairs, pair_body, 0, unroll=False)
        pltpu.make_async_copy(outs[0], qf_hbm.at[pl.ds(0, CH)], osems[0]).wait()
        pltpu.make_async_copy(outs[1], qf_hbm.at[pl.ds(0, CH)], osems[1]).wait()

    return k


def _pos_body(wuxy_ref, win_ref, pfwT_ref, pfb_ref, plg_ref, plb_ref, qpf_ref):
    ux = wuxy_ref[:, 0:1]
    uy = wuxy_ref[:, 1:2]
    has = (win_ref[...] >= 0).astype(jnp.float32)
    refx = jax.nn.sigmoid(ux)
    refy = jax.nn.sigmoid(uy)
    k = jax.lax.broadcasted_iota(jnp.int32, (1, C), 1)
    j = (k & 127) >> 1
    tdiv = jnp.exp((2.0 * j.astype(jnp.float32) / 128.0) * LOG1E4)
    pxy = jnp.where(k < 128, refx * (2.0 * math.pi), refy * (2.0 * math.pi))
    ang = pxy / tdiv
    even = (k & 1) == 0
    posf = jnp.where(even, jnp.sin(ang), jnp.cos(ang))
    pn = jnp.dot(posf, pfwT_ref[...], preferred_element_type=jnp.float32) + pfb_ref[...]
    m2 = jnp.mean(pn, axis=1, keepdims=True)
    v2 = jnp.mean((pn - m2) ** 2, axis=1, keepdims=True)
    qpf_ref[...] = ((pn - m2) / jnp.sqrt(v2 + 1e-5) * plg_ref[...] + plb_ref[...]) * has


def _pos_call(wuxy, win_col, pos_fc_w, pos_fc_b, pos_ln_g, pos_ln_b):
    full = lambda *shape: pl.BlockSpec(shape, lambda t: (0,) * len(shape))
    return pl.pallas_call(
        _pos_body,
        grid=(N // TT,),
        in_specs=[
            pl.BlockSpec((TT, 2), lambda t: (t, 0)),
            pl.BlockSpec((TT, 1), lambda t: (t, 0)),
            full(C, C), full(1, C), full(1, C), full(1, C),
        ],
        out_specs=pl.BlockSpec((TT, C), lambda t: (t, 0)),
        out_shape=jax.ShapeDtypeStruct((N, C), jnp.float32),
    )(wuxy, win_col, pos_fc_w.T, pos_fc_b.reshape(1, C),
      pos_ln_g.reshape(1, C), pos_ln_b.reshape(1, C))


_RR, _LL = B * 128, 128


def _sort_body(key_ref, idx_out_ref):
    k = key_ref[...]
    row = jax.lax.broadcasted_iota(jnp.int32, (_RR, _LL), 0)
    lane = jax.lax.broadcasted_iota(jnp.int32, (_RR, _LL), 1)
    tok = ((row & 127) << 7) | lane
    idx = tok

    def partner(x, d):
        if d >= _LL:
            dr = d // _LL
            up = jnp.concatenate([x[dr:], x[:dr]], axis=0)
            dn = jnp.concatenate([x[-dr:], x[:-dr]], axis=0)
        else:
            up = jnp.concatenate([x[:, d:], x[:, :d]], axis=1)
            dn = jnp.concatenate([x[:, -d:], x[:, :-d]], axis=1)
        return up, dn

    bs = 2
    while bs <= HW:
        d = bs // 2
        while d >= 1:
            ku, kd = partner(k, d)
            iu, jd = partner(idx, d)
            low = (tok & d) == 0
            kp = jnp.where(low, ku, kd)
            ip = jnp.where(low, iu, jd)
            a_prec = (k > kp) | ((k == kp) & (idx < ip))
            dir_desc = (tok & bs) == 0
            keep = low == (a_prec == dir_desc)
            k = jnp.where(keep, k, kp)
            idx = jnp.where(keep, idx, ip)
            d //= 2
        bs *= 2
    idx_out_ref[...] = idx


def _sort_call(keys):
    out = pl.pallas_call(
        _sort_body,
        grid=(1,),
        in_specs=[pl.BlockSpec((_RR, _LL), lambda i: (0, 0))],
        out_specs=pl.BlockSpec((_RR, _LL), lambda i: (0, 0)),
        out_shape=jax.ShapeDtypeStruct((_RR, _LL), jnp.int32),
    )(keys.reshape(_RR, _LL))
    return out.reshape(B, HW)


def _oracle_scores(cat4d, conv_w, conv_b, mem_fc_w, mem_fc_b, mem_ln_g, mem_ln_b,
                   cls_w, cls_b, invalid):
    esu = jnp.einsum('bchw,oc->bohw', cat4d, conv_w) + conv_b[None, :, None, None]
    output_memory = jnp.transpose(esu.reshape(B, C, HW), (0, 2, 1))
    output_memory = jnp.where(invalid, 0.0, output_memory)
    om = output_memory @ mem_fc_w.T + mem_fc_b
    m = jnp.mean(om, axis=-1, keepdims=True)
    v = jnp.var(om, axis=-1, keepdims=True)
    om = (om - m) / jnp.sqrt(v + 1e-5) * mem_ln_g + mem_ln_b
    cls = om @ cls_w.T + cls_b
    return jax.nn.softmax(cls, axis=-1)[..., 1]


def kernel(encode_src, feat_4x, mask, conv_w, conv_b, mem_fc_w, mem_fc_b,
           mem_ln_g, mem_ln_b, cls_w, cls_b, mlp_w1, mlp_b1, mlp_w2, mlp_b2,
           mlp_w3, mlp_b3, pos_fc_w, pos_fc_b, pos_ln_g, pos_ln_b):
    up = jnp.repeat(jnp.repeat(encode_src, 2, axis=2), 2, axis=3)
    cat4d = jnp.concatenate([up, feat_4x], axis=1)
    cat = cat4d.reshape(B, 2 * C, HW)

    esu_t, small, meta = _dense_call(
        cat, conv_w, conv_b, mem_fc_w, mem_fc_b, mem_ln_g, mem_ln_b,
        cls_w, cls_b, mlp_w1, mlp_b1, mlp_w2, mlp_b2, mlp_w3, mlp_b3)

    prop_np = _prop_table()
    validv = jnp.asarray(prop_np[:, 0] < 1e5)
    scores = _oracle_scores(cat4d, conv_w, conv_b, mem_fc_w, mem_fc_b,
                            mem_ln_g, mem_ln_b, cls_w, cls_b,
                            ~validv[None, :, None])

    iota = jnp.arange(HW, dtype=jnp.int32)
    keys = jax.lax.bitcast_convert_type(scores, jnp.int32)
    srt = _sort_call(keys)
    bidx = jnp.arange(B)[:, None]
    rank = jnp.zeros((B, HW), jnp.int32).at[bidx, srt].set(
        jnp.broadcast_to(iota[None, :], (B, HW)), unique_indices=True)
    sel = rank < K

    refx = small[..., 2]
    refy = small[..., 3]
    cls_out = small[..., 0:2]
    coord_out = jnp.stack([refy, refx], axis=-1)

    ref_all = jnp.stack([refx, refy], axis=-1)
    reference_points = jnp.take_along_axis(ref_all, srt[:, :K, None], axis=1)

    rp_x = jnp.round(refx * W).astype(jnp.int32)
    rp_y = jnp.round(refy * H).astype(jnp.int32)
    pos_idx = jnp.clip(rp_y * W + rp_x, 0, HW - 1)
    packed = jnp.where(sel, (rank << 14) | iota[None, :], -1)
    win = jnp.full((B, HW), -1, jnp.int32).at[bidx, pos_idx].max(packed)
    has = win >= 0
    wtok = jnp.where(has, win & (HW - 1), 0)

    wmeta = jnp.take_along_axis(meta, wtok[..., None], axis=1)
    hasf = has[..., None].astype(jnp.float32)
    boff = (jnp.arange(B, dtype=jnp.float32) * HW)[:, None, None]
    wmeta = jnp.concatenate(
        [wmeta[..., 0:4] + boff, wmeta[..., 4:8] * hasf, wmeta[..., 8:16]], axis=-1)
    nbr = wmeta[..., 0:4].astype(jnp.int32)
    nbrflat = jnp.transpose(nbr.reshape(N // CH, CH, 4), (0, 2, 1)).reshape(4 * N)

    place = _make_placement()
    qf = place(esu_t.reshape(N, C), wmeta.reshape(N, 16), nbrflat)

    wuxy = wmeta[..., 8:10].reshape(N, 2)
    qpf = _pos_call(wuxy, win.reshape(N, 1), pos_fc_w, pos_fc_b, pos_ln_g, pos_ln_b)

    query = jnp.transpose(qf.reshape(B, HW, C), (0, 2, 1)).reshape(B, C, H, W)
    query_pos = jnp.transpose(qpf.reshape(B, HW, C), (1, 0, 2))
    return (query, query_pos, reference_points, cls_out, coord_out)
